# Initial kernel scaffold; baseline (speedup 1.0000x reference)
import jax
import jax.numpy as jnp
from jax.experimental import pallas as pl


def kernel(user, traj, geo, long_traj, traj_graph_x, geo_graph_x, user_table, loc_table, geo_table):
    raise NotImplementedError("write your pallas kernel here")


# speedup vs baseline: 1.2219x; 1.2219x over previous
import functools

import jax
import jax.numpy as jnp
from jax import lax
from jax.experimental import pallas as pl
from jax.experimental.pallas import tpu as pltpu
from jax.experimental.pallas import tpu_sc as plsc

HIDDEN = 64
CHUNK = 128

_info = plsc.get_sparse_core_info()
NC, NS = _info.num_cores, _info.num_subcores
NW = NC * NS


def _run_job(idx_hbm, tab_hbm, out_hbm, nchunks, base, idx_v, rows_v, sem):
    def step(c, carry):
        off = base + c * CHUNK
        pltpu.sync_copy(idx_hbm.at[pl.ds(off, CHUNK)], idx_v)
        pltpu.async_copy(tab_hbm.at[idx_v], rows_v, sem).wait()
        pltpu.sync_copy(rows_v, out_hbm.at[pl.ds(off, CHUNK)])
        return carry

    lax.fori_loop(0, nchunks, step, 0)


def _sc_kernel(n_traj, n_graph_pad,
               user_i, traj_i, geo_i, ltraj_i, tgx_i, ggx_i,
               user_t, loc_t, geo_t,
               o_user, o_traj, o_geo, o_ltraj, o_tgx, o_ggx,
               idx_v, rows_v, sem):
    wid = lax.axis_index("s") * NC + lax.axis_index("c")
    jobs = [
        (user_i, user_t, o_user, 4096 // (NW * CHUNK)),
        (traj_i, loc_t, o_traj, n_traj // (NW * CHUNK)),
        (geo_i, geo_t, o_geo, n_traj // (NW * CHUNK)),
        (ltraj_i, loc_t, o_ltraj, n_traj // (NW * CHUNK)),
        (tgx_i, loc_t, o_tgx, n_graph_pad // (NW * CHUNK)),
        (ggx_i, geo_t, o_ggx, n_graph_pad // (NW * CHUNK)),
    ]
    for idx_hbm, tab, out, nchunks in jobs:
        base = wid * nchunks * CHUNK
        _run_job(idx_hbm, tab, out, nchunks, base, idx_v, rows_v, sem)


def kernel(user, traj, geo, long_traj, traj_graph_x, geo_graph_x,
           user_table, loc_table, geo_table):
    n_traj = traj.shape[0] * traj.shape[1]
    n_graph = traj_graph_x.shape[0]
    gran = NW * CHUNK
    n_graph_pad = -(-n_graph // gran) * gran

    traj_f = traj.reshape(-1)
    geo_f = geo.reshape(-1)
    ltraj_f = long_traj.reshape(-1)
    tgx_p = jnp.pad(traj_graph_x, (0, n_graph_pad - n_graph))
    ggx_p = jnp.pad(geo_graph_x, (0, n_graph_pad - n_graph))

    mesh = plsc.VectorSubcoreMesh(core_axis_name="c", subcore_axis_name="s")
    f = pl.kernel(
        functools.partial(_sc_kernel, n_traj, n_graph_pad),
        mesh=mesh,
        compiler_params=pltpu.CompilerParams(use_tc_tiling_on_sc=False),
        out_type=[
            jax.ShapeDtypeStruct((4096, HIDDEN), jnp.float32),
            jax.ShapeDtypeStruct((n_traj, HIDDEN), jnp.float32),
            jax.ShapeDtypeStruct((n_traj, HIDDEN), jnp.float32),
            jax.ShapeDtypeStruct((n_traj, HIDDEN), jnp.float32),
            jax.ShapeDtypeStruct((n_graph_pad, HIDDEN), jnp.float32),
            jax.ShapeDtypeStruct((n_graph_pad, HIDDEN), jnp.float32),
        ],
        scratch_types=[
            pltpu.VMEM((CHUNK,), jnp.int32),
            pltpu.VMEM((CHUNK, HIDDEN), jnp.float32),
            pltpu.SemaphoreType.DMA,
        ],
    )
    o_user, o_traj, o_geo, o_ltraj, o_tgx, o_ggx = f(
        user, traj_f, geo_f, ltraj_f, tgx_p, ggx_p,
        user_table, loc_table, geo_table)
    B, T = traj.shape
    return (
        o_user,
        o_traj.reshape(B, T, HIDDEN),
        o_geo.reshape(B, T, HIDDEN),
        o_ltraj.reshape(B, T, HIDDEN),
        o_tgx[:n_graph],
        o_ggx[:n_graph],
    )


# speedup vs baseline: 1.4689x; 1.2021x over previous
import functools

import jax
import jax.numpy as jnp
from jax import lax
from jax.experimental import pallas as pl
from jax.experimental.pallas import tpu as pltpu
from jax.experimental.pallas import tpu_sc as plsc

HIDDEN = 64
CHUNK = 128

_info = plsc.get_sparse_core_info()
NC, NS = _info.num_cores, _info.num_subcores
NW = NC * NS


NBUF = 4


def _run_job(idx_hbm, tab_hbm, out_hbm, nchunks, base, idx_v, rows_v, sems):
    nb = min(NBUF, nchunks)
    for b in range(nb):
        off = base + b * CHUNK
        pltpu.sync_copy(idx_hbm.at[pl.ds(off, CHUNK)], idx_v.at[b])
        pltpu.async_copy(tab_hbm.at[idx_v.at[b]], rows_v.at[b], sems[b])

    ngroups = -(-nchunks // NBUF)

    def group(g, carry):
        for b in range(NBUF):
            c = g * NBUF + b

            @pl.when(c < nchunks)
            def _():
                pltpu.make_async_copy(
                    tab_hbm.at[idx_v.at[b]], rows_v.at[b], sems[b]).wait()
                off = base + c * CHUNK
                pltpu.sync_copy(rows_v.at[b], out_hbm.at[pl.ds(off, CHUNK)])

                @pl.when(c + NBUF < nchunks)
                def _():
                    off2 = base + (c + NBUF) * CHUNK
                    pltpu.sync_copy(idx_hbm.at[pl.ds(off2, CHUNK)], idx_v.at[b])
                    pltpu.async_copy(
                        tab_hbm.at[idx_v.at[b]], rows_v.at[b], sems[b])

        return carry

    lax.fori_loop(0, ngroups, group, 0)


def _sc_kernel(n_traj, n_graph_pad,
               user_i, traj_i, geo_i, ltraj_i, tgx_i, ggx_i,
               user_t, loc_t, geo_t,
               o_user, o_traj, o_geo, o_ltraj, o_tgx, o_ggx,
               idx_v, rows_v, *sems):
    wid = lax.axis_index("s") * NC + lax.axis_index("c")
    jobs = [
        (user_i, user_t, o_user, 4096 // (NW * CHUNK)),
        (traj_i, loc_t, o_traj, n_traj // (NW * CHUNK)),
        (geo_i, geo_t, o_geo, n_traj // (NW * CHUNK)),
        (ltraj_i, loc_t, o_ltraj, n_traj // (NW * CHUNK)),
        (tgx_i, loc_t, o_tgx, n_graph_pad // (NW * CHUNK)),
        (ggx_i, geo_t, o_ggx, n_graph_pad // (NW * CHUNK)),
    ]
    for idx_hbm, tab, out, nchunks in jobs:
        base = wid * nchunks * CHUNK
        _run_job(idx_hbm, tab, out, nchunks, base, idx_v, rows_v, sems)


def kernel(user, traj, geo, long_traj, traj_graph_x, geo_graph_x,
           user_table, loc_table, geo_table):
    n_traj = traj.shape[0] * traj.shape[1]
    n_graph = traj_graph_x.shape[0]
    gran = NW * CHUNK
    n_graph_pad = -(-n_graph // gran) * gran

    traj_f = traj.reshape(-1)
    geo_f = geo.reshape(-1)
    ltraj_f = long_traj.reshape(-1)
    tgx_p = jnp.pad(traj_graph_x, (0, n_graph_pad - n_graph))
    ggx_p = jnp.pad(geo_graph_x, (0, n_graph_pad - n_graph))

    mesh = plsc.VectorSubcoreMesh(core_axis_name="c", subcore_axis_name="s")
    f = pl.kernel(
        functools.partial(_sc_kernel, n_traj, n_graph_pad),
        mesh=mesh,
        compiler_params=pltpu.CompilerParams(use_tc_tiling_on_sc=False),
        out_type=[
            jax.ShapeDtypeStruct((4096, HIDDEN), jnp.float32),
            jax.ShapeDtypeStruct((n_traj, HIDDEN), jnp.float32),
            jax.ShapeDtypeStruct((n_traj, HIDDEN), jnp.float32),
            jax.ShapeDtypeStruct((n_traj, HIDDEN), jnp.float32),
            jax.ShapeDtypeStruct((n_graph_pad, HIDDEN), jnp.float32),
            jax.ShapeDtypeStruct((n_graph_pad, HIDDEN), jnp.float32),
        ],
        scratch_types=[
            pltpu.VMEM((NBUF, CHUNK), jnp.int32),
            pltpu.VMEM((NBUF, CHUNK, HIDDEN), jnp.float32),
        ] + [pltpu.SemaphoreType.DMA] * NBUF,
    )
    o_user, o_traj, o_geo, o_ltraj, o_tgx, o_ggx = f(
        user, traj_f, geo_f, ltraj_f, tgx_p, ggx_p,
        user_table, loc_table, geo_table)
    B, T = traj.shape
    return (
        o_user,
        o_traj.reshape(B, T, HIDDEN),
        o_geo.reshape(B, T, HIDDEN),
        o_ltraj.reshape(B, T, HIDDEN),
        o_tgx[:n_graph],
        o_ggx[:n_graph],
    )


# speedup vs baseline: 1.5556x; 1.0590x over previous
import functools

import jax
import jax.numpy as jnp
from jax import lax
from jax.experimental import pallas as pl
from jax.experimental.pallas import tpu as pltpu
from jax.experimental.pallas import tpu_sc as plsc

HIDDEN = 64
CHUNK = 128
NBUF = 8

_info = plsc.get_sparse_core_info()
NC, NS = _info.num_cores, _info.num_subcores
NW = NC * NS


def _run_job(idx_hbm, tab_hbm, out_hbm, nchunks, wid, idx_v, rows_v,
             sems_g, sems_s):
    chunk0 = wid * nchunks
    nb = min(NBUF, nchunks)

    pltpu.sync_copy(idx_hbm.at[pl.ds(chunk0, nchunks)],
                    idx_v.at[pl.ds(0, nchunks)])

    def fire_gather(b, c):
        pltpu.async_copy(tab_hbm.at[idx_v.at[c]], rows_v.at[b], sems_g[b])

    def wait_gather(b):
        pltpu.make_async_copy(tab_hbm.at[idx_v.at[0]], rows_v.at[b],
                              sems_g[b]).wait()

    def fire_store(b, c):
        pltpu.async_copy(rows_v.at[b],
                         out_hbm.at[pl.ds((chunk0 + c) * CHUNK, CHUNK)],
                         sems_s[b])

    def wait_store(b):
        pltpu.make_async_copy(rows_v.at[b],
                              out_hbm.at[pl.ds(0, CHUNK)], sems_s[b]).wait()

    for b in range(nb):
        fire_gather(b, b)

    def group(g, carry):
        for b in range(NBUF):
            c = g * NBUF + b

            @pl.when(c < nchunks)
            def _():
                wait_gather(b)
                fire_store(b, c)

        for b in range(NBUF):
            c = g * NBUF + b

            @pl.when(c + NBUF < nchunks)
            def _():
                wait_store(b)
                fire_gather(b, c + NBUF)

        return carry

    lax.fori_loop(0, -(-nchunks // NBUF), group, 0)

    for b in range(nb):
        wait_store(b)


def _sc_kernel(n_traj, n_graph_pad,
               user_i, traj_i, geo_i, ltraj_i, tgx_i, ggx_i,
               user_t, loc_t, geo_t,
               o_user, o_traj, o_geo, o_ltraj, o_tgx, o_ggx,
               idx_v, rows_v, *sems):
    wid = lax.axis_index("s") * NC + lax.axis_index("c")
    sems_g, sems_s = sems[:NBUF], sems[NBUF:]
    jobs = [
        (user_i, user_t, o_user, 4096 // (NW * CHUNK)),
        (traj_i, loc_t, o_traj, n_traj // (NW * CHUNK)),
        (geo_i, geo_t, o_geo, n_traj // (NW * CHUNK)),
        (ltraj_i, loc_t, o_ltraj, n_traj // (NW * CHUNK)),
        (tgx_i, loc_t, o_tgx, n_graph_pad // (NW * CHUNK)),
        (ggx_i, geo_t, o_ggx, n_graph_pad // (NW * CHUNK)),
    ]
    for idx_hbm, tab, out, nchunks in jobs:
        _run_job(idx_hbm, tab, out, nchunks, wid, idx_v, rows_v,
                 sems_g, sems_s)


def kernel(user, traj, geo, long_traj, traj_graph_x, geo_graph_x,
           user_table, loc_table, geo_table):
    n_traj = traj.shape[0] * traj.shape[1]
    n_graph = traj_graph_x.shape[0]
    gran = NW * CHUNK
    n_graph_pad = -(-n_graph // gran) * gran
    max_chunks = n_traj // (NW * CHUNK)

    user_c = user.reshape(-1, CHUNK)
    traj_c = traj.reshape(-1, CHUNK)
    geo_c = geo.reshape(-1, CHUNK)
    ltraj_c = long_traj.reshape(-1, CHUNK)
    tgx_c = jnp.pad(traj_graph_x, (0, n_graph_pad - n_graph)).reshape(-1, CHUNK)
    ggx_c = jnp.pad(geo_graph_x, (0, n_graph_pad - n_graph)).reshape(-1, CHUNK)

    mesh = plsc.VectorSubcoreMesh(core_axis_name="c", subcore_axis_name="s")
    f = pl.kernel(
        functools.partial(_sc_kernel, n_traj, n_graph_pad),
        mesh=mesh,
        compiler_params=pltpu.CompilerParams(use_tc_tiling_on_sc=False),
        out_type=[
            jax.ShapeDtypeStruct((4096, HIDDEN), jnp.float32),
            jax.ShapeDtypeStruct((n_traj, HIDDEN), jnp.float32),
            jax.ShapeDtypeStruct((n_traj, HIDDEN), jnp.float32),
            jax.ShapeDtypeStruct((n_traj, HIDDEN), jnp.float32),
            jax.ShapeDtypeStruct((n_graph_pad, HIDDEN), jnp.float32),
            jax.ShapeDtypeStruct((n_graph_pad, HIDDEN), jnp.float32),
        ],
        scratch_types=[
            pltpu.VMEM((max_chunks, CHUNK), jnp.int32),
            pltpu.VMEM((NBUF, CHUNK, HIDDEN), jnp.float32),
        ] + [pltpu.SemaphoreType.DMA] * (2 * NBUF),
    )
    o_user, o_traj, o_geo, o_ltraj, o_tgx, o_ggx = f(
        user_c, traj_c, geo_c, ltraj_c, tgx_c, ggx_c,
        user_table, loc_table, geo_table)
    B, T = traj.shape
    return (
        o_user,
        o_traj.reshape(B, T, HIDDEN),
        o_geo.reshape(B, T, HIDDEN),
        o_ltraj.reshape(B, T, HIDDEN),
        o_tgx[:n_graph],
        o_ggx[:n_graph],
    )
